# baseline (speedup 1.0000x reference)
import functools

import jax
import jax.numpy as jnp
from jax.experimental import pallas as pl

B, S, H = 1, 2048, 1024
N, K = 8, 2
R = 2
V = 32000
I = 128
SCALE = 2.0

BT = 256
BN = 640

_EPS = jnp.finfo(jnp.float32).eps


def _dot_t(a, b):
    return jax.lax.dot_general(a, b, (((1,), (1,)), ((), ())),
                               preferred_element_type=jnp.float32)


def _dot_t_bf16(a, b):
    return jax.lax.dot_general(a.astype(jnp.bfloat16), b, (((1,), (1,)), ((), ())),
                               preferred_element_type=jnp.float32)


def _pre_kernel(x_ref, Acat_ref, Bcat_ref, oA_ref, oB_ref, n1w_ref, n2w_ref,
                Wr_ref, gW_ref, gb_ref, uW_ref, ub_ref, x2_ref):
    xa = x_ref[...]
    t = _dot_t(xa, Acat_ref[...])
    s = jax.lax.dot_general(t, Bcat_ref[...], (((1,), (1,)), ((), ())),
                            preferred_element_type=jnp.float32) * SCALE
    t2 = _dot_t(s, oA_ref[...])
    a = jax.lax.dot_general(t2, oB_ref[...], (((1,), (1,)), ((), ())),
                            preferred_element_type=jnp.float32) * SCALE
    h1 = xa + a
    ms1 = jnp.mean(h1 * h1, axis=-1, keepdims=True)
    x1 = h1 * jax.lax.rsqrt(ms1 + _EPS) * n1w_ref[...]

    logits = _dot_t(x1, Wr_ref[...])
    idx = jax.lax.broadcasted_iota(jnp.int32, (BT, N), 1)
    m1 = jnp.max(logits, axis=-1, keepdims=True)
    e1 = jnp.min(jnp.where(logits == m1, idx, N), axis=-1, keepdims=True)
    l2 = jnp.where(idx == e1, -jnp.inf, logits)
    m2 = jnp.max(l2, axis=-1, keepdims=True)
    e2 = jnp.min(jnp.where(l2 == m2, idx, N), axis=-1, keepdims=True)
    p = jnp.exp(m2 - m1)
    w1 = 1.0 / (1.0 + p)
    w2 = p * w1
    w = jnp.where(idx == e1, w1, 0.0) + jnp.where(idx == e2, w2, 0.0)

    x1b = x1.astype(jnp.bfloat16)
    acc = jnp.zeros((BT, H), jnp.float32)
    for e in range(N):
        g = jax.lax.dot_general(x1b, gW_ref[e], (((1,), (1,)), ((), ())),
                                preferred_element_type=jnp.float32) + gb_ref[e]
        act = g * jax.nn.sigmoid(g)
        eo = _dot_t_bf16(act * w[:, e:e + 1], uW_ref[e]) + w[:, e:e + 1] * ub_ref[e]
        acc = acc + eo

    h2 = x1 + acc
    ms2 = jnp.mean(h2 * h2, axis=-1, keepdims=True)
    x2 = h2 * jax.lax.rsqrt(ms2 + _EPS) * n2w_ref[...]
    x2_ref[...] = x2.astype(jnp.bfloat16)


def _head_kernel(x2_ref, w_ref, out_ref):
    out_ref[...] = _dot_t(x2_ref[...], w_ref[...])


def _full(shape):
    nd = len(shape)
    return pl.BlockSpec(shape, lambda i: (0,) * nd)


def kernel(x, qA, qB, kA, kB, vA, vB, oA, oB, n1w, n2w, Wr, gW, gb, uW, ub, headW):
    xf = x.reshape(S, H)
    Acat = jnp.concatenate([qA, kA, vA], axis=0)
    Bcat = jnp.concatenate([qB, kB, vB], axis=1)
    n1w2 = n1w.reshape(1, H)
    n2w2 = n2w.reshape(1, H)
    gWb = gW.astype(jnp.bfloat16)
    uWb = uW.astype(jnp.bfloat16)
    headWb = headW.astype(jnp.bfloat16)

    x2 = pl.pallas_call(
        _pre_kernel,
        grid=(S // BT,),
        in_specs=[
            pl.BlockSpec((BT, H), lambda i: (i, 0)),
            _full((6, H)), _full((H, 6)), _full((R, H)), _full((H, R)),
            _full((1, H)), _full((1, H)), _full((N, H)),
            _full((N, I, H)), _full((N, I)), _full((N, H, I)), _full((N, H)),
        ],
        out_specs=pl.BlockSpec((BT, H), lambda i: (i, 0)),
        out_shape=jax.ShapeDtypeStruct((S, H), jnp.bfloat16),
    )(xf, Acat, Bcat, oA, oB, n1w2, n2w2, Wr, gWb, gb, uWb, ub)

    out = pl.pallas_call(
        _head_kernel,
        grid=(V // BN,),
        in_specs=[
            pl.BlockSpec((S, H), lambda j: (0, 0)),
            pl.BlockSpec((BN, H), lambda j: (j, 0)),
        ],
        out_specs=pl.BlockSpec((S, BN), lambda j: (0, j)),
        out_shape=jax.ShapeDtypeStruct((S, V), jnp.float32),
    )(x2, headWb)

    return out.reshape(B, S, V)


# speedup vs baseline: 1.2378x; 1.2378x over previous
import functools

import jax
import jax.numpy as jnp
from jax.experimental import pallas as pl

B, S, H = 1, 2048, 1024
N, K = 8, 2
R = 2
V = 32000
I = 128
SCALE = 2.0

BT = 256
BN = 640

_EPS = jnp.finfo(jnp.float32).eps


def _dot_t(a, b):
    return jax.lax.dot_general(a, b, (((1,), (1,)), ((), ())),
                               preferred_element_type=jnp.float32)


def _dot_t_bf16(a, b):
    return jax.lax.dot_general(a.astype(jnp.bfloat16), b, (((1,), (1,)), ((), ())),
                               preferred_element_type=jnp.float32)


def _pre_kernel(x_ref, Acat_ref, Bcat_ref, oA_ref, oB_ref, n1w_ref, n2w_ref,
                Wr_ref, gW_ref, gb_ref, uW_ref, ub_ref, x2_ref):
    xa = x_ref[...]
    t = _dot_t(xa, Acat_ref[...])
    s = jax.lax.dot_general(t, Bcat_ref[...], (((1,), (1,)), ((), ())),
                            preferred_element_type=jnp.float32) * SCALE
    t2 = _dot_t(s, oA_ref[...])
    a = jax.lax.dot_general(t2, oB_ref[...], (((1,), (1,)), ((), ())),
                            preferred_element_type=jnp.float32) * SCALE
    h1 = xa + a
    ms1 = jnp.mean(h1 * h1, axis=-1, keepdims=True)
    x1 = h1 * jax.lax.rsqrt(ms1 + _EPS) * n1w_ref[...]

    logits = _dot_t(x1, Wr_ref[...])
    idx = jax.lax.broadcasted_iota(jnp.int32, (BT, N), 1)
    m1 = jnp.max(logits, axis=-1, keepdims=True)
    e1 = jnp.min(jnp.where(logits == m1, idx, N), axis=-1, keepdims=True)
    l2 = jnp.where(idx == e1, -jnp.inf, logits)
    m2 = jnp.max(l2, axis=-1, keepdims=True)
    e2 = jnp.min(jnp.where(l2 == m2, idx, N), axis=-1, keepdims=True)
    p = jnp.exp(m2 - m1)
    w1 = 1.0 / (1.0 + p)
    w2 = p * w1
    w = jnp.where(idx == e1, w1, 0.0) + jnp.where(idx == e2, w2, 0.0)

    x1b = x1.astype(jnp.bfloat16)
    acc = jnp.zeros((BT, H), jnp.float32)
    for e in range(N):
        g = jax.lax.dot_general(x1b, gW_ref[e], (((1,), (1,)), ((), ())),
                                preferred_element_type=jnp.float32) + gb_ref[e]
        act = g * jax.nn.sigmoid(g)
        eo = _dot_t_bf16(act * w[:, e:e + 1], uW_ref[e]) + w[:, e:e + 1] * ub_ref[e]
        acc = acc + eo

    h2 = x1 + acc
    ms2 = jnp.mean(h2 * h2, axis=-1, keepdims=True)
    x2 = h2 * jax.lax.rsqrt(ms2 + _EPS) * n2w_ref[...]
    x2_ref[...] = x2.astype(jnp.bfloat16)


def _head_kernel(x2_ref, w_ref, out_ref):
    out_ref[...] = jax.lax.dot_general(
        x2_ref[...], w_ref[...].astype(jnp.bfloat16),
        (((1,), (1,)), ((), ())), preferred_element_type=jnp.float32)


def _full(shape):
    nd = len(shape)
    return pl.BlockSpec(shape, lambda i: (0,) * nd)


def kernel(x, qA, qB, kA, kB, vA, vB, oA, oB, n1w, n2w, Wr, gW, gb, uW, ub, headW):
    xf = x.reshape(S, H)
    Acat = jnp.concatenate([qA, kA, vA], axis=0)
    Bcat = jnp.concatenate([qB, kB, vB], axis=1)
    n1w2 = n1w.reshape(1, H)
    n2w2 = n2w.reshape(1, H)
    gWb = gW.astype(jnp.bfloat16)
    uWb = uW.astype(jnp.bfloat16)

    x2 = pl.pallas_call(
        _pre_kernel,
        grid=(S // BT,),
        in_specs=[
            pl.BlockSpec((BT, H), lambda i: (i, 0)),
            _full((6, H)), _full((H, 6)), _full((R, H)), _full((H, R)),
            _full((1, H)), _full((1, H)), _full((N, H)),
            _full((N, I, H)), _full((N, I)), _full((N, H, I)), _full((N, H)),
        ],
        out_specs=pl.BlockSpec((BT, H), lambda i: (i, 0)),
        out_shape=jax.ShapeDtypeStruct((S, H), jnp.bfloat16),
    )(xf, Acat, Bcat, oA, oB, n1w2, n2w2, Wr, gWb, gb, uWb, ub)

    out = pl.pallas_call(
        _head_kernel,
        grid=(V // BN,),
        in_specs=[
            pl.BlockSpec((S, H), lambda j: (0, 0)),
            pl.BlockSpec((BN, H), lambda j: (j, 0)),
        ],
        out_specs=pl.BlockSpec((S, BN), lambda j: (0, j)),
        out_shape=jax.ShapeDtypeStruct((S, V), jnp.float32),
    )(x2, headW)

    return out.reshape(B, S, V)


# speedup vs baseline: 1.3631x; 1.1012x over previous
import functools

import jax
import jax.numpy as jnp
from jax.experimental import pallas as pl

B, S, H = 1, 2048, 1024
N, K = 8, 2
R = 2
V = 32000
I = 128
SCALE = 2.0

BT = 256
BN = 640

_EPS = jnp.finfo(jnp.float32).eps


def _dot_t(a, b):
    return jax.lax.dot_general(a, b, (((1,), (1,)), ((), ())),
                               preferred_element_type=jnp.float32)


def _dot_t_bf16(a, b):
    return jax.lax.dot_general(a.astype(jnp.bfloat16), b, (((1,), (1,)), ((), ())),
                               preferred_element_type=jnp.float32)


def _pre_kernel(x_ref, Acat_ref, Bcat_ref, oA_ref, oB_ref, n1w_ref, n2w_ref,
                Wr_ref, gcat_ref, gbf_ref, ucat_ref, ub_ref, x2_ref):
    xa = x_ref[...]
    t = _dot_t(xa, Acat_ref[...])
    s = jax.lax.dot_general(t, Bcat_ref[...], (((1,), (1,)), ((), ())),
                            preferred_element_type=jnp.float32) * SCALE
    t2 = _dot_t(s, oA_ref[...])
    a = jax.lax.dot_general(t2, oB_ref[...], (((1,), (1,)), ((), ())),
                            preferred_element_type=jnp.float32) * SCALE
    h1 = xa + a
    ms1 = jnp.mean(h1 * h1, axis=-1, keepdims=True)
    x1 = h1 * jax.lax.rsqrt(ms1 + _EPS) * n1w_ref[...]

    logits = _dot_t(x1, Wr_ref[...])
    idx = jax.lax.broadcasted_iota(jnp.int32, (BT, N), 1)
    m1 = jnp.max(logits, axis=-1, keepdims=True)
    e1 = jnp.min(jnp.where(logits == m1, idx, N), axis=-1, keepdims=True)
    l2 = jnp.where(idx == e1, -jnp.inf, logits)
    m2 = jnp.max(l2, axis=-1, keepdims=True)
    e2 = jnp.min(jnp.where(l2 == m2, idx, N), axis=-1, keepdims=True)
    p = jnp.exp(m2 - m1)
    w1 = 1.0 / (1.0 + p)
    w2 = p * w1
    w = jnp.where(idx == e1, w1, 0.0) + jnp.where(idx == e2, w2, 0.0)

    x1b = x1.astype(jnp.bfloat16)
    g = jax.lax.dot_general(x1b, gcat_ref[...], (((1,), (1,)), ((), ())),
                            preferred_element_type=jnp.float32) + gbf_ref[...]
    act = g * jax.nn.sigmoid(g)
    w_exp = jnp.repeat(w, I, axis=1)
    actw = (act * w_exp).astype(jnp.bfloat16)
    acc = jax.lax.dot_general(actw, ucat_ref[...], (((1,), (0,)), ((), ())),
                              preferred_element_type=jnp.float32)
    acc = acc + jax.lax.dot_general(w, ub_ref[...], (((1,), (0,)), ((), ())),
                                    preferred_element_type=jnp.float32)

    h2 = x1 + acc
    ms2 = jnp.mean(h2 * h2, axis=-1, keepdims=True)
    x2_ref[...] = h2 * jax.lax.rsqrt(ms2 + _EPS) * n2w_ref[...]


def _head_kernel(x2_ref, w_ref, out_ref):
    out_ref[...] = jax.lax.dot_general(
        x2_ref[...], w_ref[...],
        (((1,), (1,)), ((), ())), preferred_element_type=jnp.float32)


def _full(shape):
    nd = len(shape)
    return pl.BlockSpec(shape, lambda i: (0,) * nd)


def kernel(x, qA, qB, kA, kB, vA, vB, oA, oB, n1w, n2w, Wr, gW, gb, uW, ub, headW):
    xf = x.reshape(S, H)
    Acat = jnp.concatenate([qA, kA, vA], axis=0)
    Bcat = jnp.concatenate([qB, kB, vB], axis=1)
    n1w2 = n1w.reshape(1, H)
    n2w2 = n2w.reshape(1, H)
    gcat = gW.reshape(N * I, H).astype(jnp.bfloat16)
    gbf = gb.reshape(1, N * I)
    ucat = jnp.transpose(uW, (0, 2, 1)).reshape(N * I, H).astype(jnp.bfloat16)

    x2 = pl.pallas_call(
        _pre_kernel,
        grid=(S // BT,),
        in_specs=[
            pl.BlockSpec((BT, H), lambda i: (i, 0)),
            _full((6, H)), _full((H, 6)), _full((R, H)), _full((H, R)),
            _full((1, H)), _full((1, H)), _full((N, H)),
            _full((N * I, H)), _full((1, N * I)), _full((N * I, H)), _full((N, H)),
        ],
        out_specs=pl.BlockSpec((BT, H), lambda i: (i, 0)),
        out_shape=jax.ShapeDtypeStruct((S, H), jnp.float32),
    )(xf, Acat, Bcat, oA, oB, n1w2, n2w2, Wr, gcat, gbf, ucat, ub)

    out = pl.pallas_call(
        _head_kernel,
        grid=(V // BN,),
        in_specs=[
            pl.BlockSpec((S, H), lambda j: (0, 0)),
            pl.BlockSpec((BN, H), lambda j: (j, 0)),
        ],
        out_specs=pl.BlockSpec((S, BN), lambda j: (0, j)),
        out_shape=jax.ShapeDtypeStruct((S, V), jnp.float32),
    )(x2, headW)

    return out.reshape(B, S, V)


# speedup vs baseline: 1.5899x; 1.1664x over previous
import functools

import jax
import jax.numpy as jnp
from jax.experimental import pallas as pl

B, S, H = 1, 2048, 1024
N, K = 8, 2
R = 2
V = 32000
I = 128
SCALE = 2.0

BT = 256
BN = 1280

_EPS = jnp.finfo(jnp.float32).eps


def _dot_t(a, b):
    return jax.lax.dot_general(a, b, (((1,), (1,)), ((), ())),
                               preferred_element_type=jnp.float32)


def _dot_t_bf16(a, b):
    return jax.lax.dot_general(a.astype(jnp.bfloat16), b, (((1,), (1,)), ((), ())),
                               preferred_element_type=jnp.float32)


def _pre_kernel(x_ref, Acat_ref, Bcat_ref, oA_ref, oB_ref, n1w_ref, n2w_ref,
                Wr_ref, gcat_ref, gbf_ref, ucat_ref, ub_ref, x2_ref):
    xa = x_ref[...]
    t = _dot_t(xa, Acat_ref[...])
    s = jax.lax.dot_general(t, Bcat_ref[...], (((1,), (1,)), ((), ())),
                            preferred_element_type=jnp.float32) * SCALE
    t2 = _dot_t(s, oA_ref[...])
    a = jax.lax.dot_general(t2, oB_ref[...], (((1,), (1,)), ((), ())),
                            preferred_element_type=jnp.float32) * SCALE
    h1 = xa + a
    ms1 = jnp.mean(h1 * h1, axis=-1, keepdims=True)
    x1 = h1 * jax.lax.rsqrt(ms1 + _EPS) * n1w_ref[...]

    logits = _dot_t(x1, Wr_ref[...])
    idx = jax.lax.broadcasted_iota(jnp.int32, (BT, N), 1)
    m1 = jnp.max(logits, axis=-1, keepdims=True)
    e1 = jnp.min(jnp.where(logits == m1, idx, N), axis=-1, keepdims=True)
    l2 = jnp.where(idx == e1, -jnp.inf, logits)
    m2 = jnp.max(l2, axis=-1, keepdims=True)
    e2 = jnp.min(jnp.where(l2 == m2, idx, N), axis=-1, keepdims=True)
    p = jnp.exp(m2 - m1)
    w1 = 1.0 / (1.0 + p)
    w2 = p * w1
    w = jnp.where(idx == e1, w1, 0.0) + jnp.where(idx == e2, w2, 0.0)

    x1b = x1.astype(jnp.bfloat16)
    g = jax.lax.dot_general(x1b, gcat_ref[...], (((1,), (1,)), ((), ())),
                            preferred_element_type=jnp.float32) + gbf_ref[...]
    act = g * jax.nn.sigmoid(g)
    w_exp = jnp.repeat(w, I, axis=1)
    actw = (act * w_exp).astype(jnp.bfloat16)
    acc = jax.lax.dot_general(actw, ucat_ref[...], (((1,), (0,)), ((), ())),
                              preferred_element_type=jnp.float32)
    acc = acc + jax.lax.dot_general(w, ub_ref[...], (((1,), (0,)), ((), ())),
                                    preferred_element_type=jnp.float32)

    h2 = x1 + acc
    ms2 = jnp.mean(h2 * h2, axis=-1, keepdims=True)
    x2_ref[...] = h2 * jax.lax.rsqrt(ms2 + _EPS) * n2w_ref[...]


def _head_kernel(x2_ref, w_ref, out_ref):
    out_ref[...] = jax.lax.dot_general(
        x2_ref[...], w_ref[...],
        (((1,), (1,)), ((), ())), preferred_element_type=jnp.float32)


def _full(shape):
    nd = len(shape)
    return pl.BlockSpec(shape, lambda i: (0,) * nd)


def kernel(x, qA, qB, kA, kB, vA, vB, oA, oB, n1w, n2w, Wr, gW, gb, uW, ub, headW):
    xf = x.reshape(S, H)
    Acat = jnp.concatenate([qA, kA, vA], axis=0)
    Bcat = jnp.concatenate([qB, kB, vB], axis=1)
    n1w2 = n1w.reshape(1, H)
    n2w2 = n2w.reshape(1, H)
    gcat = gW.reshape(N * I, H).astype(jnp.bfloat16)
    gbf = gb.reshape(1, N * I)
    ucat = jnp.transpose(uW, (0, 2, 1)).reshape(N * I, H).astype(jnp.bfloat16)

    x2 = pl.pallas_call(
        _pre_kernel,
        grid=(S // BT,),
        in_specs=[
            pl.BlockSpec((BT, H), lambda i: (i, 0)),
            _full((6, H)), _full((H, 6)), _full((R, H)), _full((H, R)),
            _full((1, H)), _full((1, H)), _full((N, H)),
            _full((N * I, H)), _full((1, N * I)), _full((N * I, H)), _full((N, H)),
        ],
        out_specs=pl.BlockSpec((BT, H), lambda i: (i, 0)),
        out_shape=jax.ShapeDtypeStruct((S, H), jnp.float32),
    )(xf, Acat, Bcat, oA, oB, n1w2, n2w2, Wr, gcat, gbf, ucat, ub)

    out = pl.pallas_call(
        _head_kernel,
        grid=(V // BN,),
        in_specs=[
            pl.BlockSpec((S, H), lambda j: (0, 0)),
            pl.BlockSpec((BN, H), lambda j: (j, 0)),
        ],
        out_specs=pl.BlockSpec((S, BN), lambda j: (0, j)),
        out_shape=jax.ShapeDtypeStruct((S, V), jnp.float32),
    )(x2, headW)

    return out.reshape(B, S, V)
